# baseline (speedup 1.0000x reference)
import functools

import jax
import jax.numpy as jnp
from jax import lax
from jax.experimental import pallas as pl
from jax.experimental.pallas import tpu as pltpu
from jax.experimental.pallas import tpu_sc as plsc

_NC, _NS = 2, 16
_NW = _NC * _NS
_CH = 128


def _sc_gather(flat_b, flat_c, idx_b, idx_c, rows, eb, ec):
    rows_w = rows // _NW
    n_ch = rows_w // _CH
    mesh = plsc.VectorSubcoreMesh(core_axis_name="c", subcore_axis_name="s")

    @functools.partial(
        pl.kernel,
        out_type=(jax.ShapeDtypeStruct((rows, eb), jnp.float32),
                  jax.ShapeDtypeStruct((rows, ec), jnp.float32)),
        mesh=mesh,
        scratch_types=[
            pltpu.VMEM((n_ch, _CH), jnp.int32),
            pltpu.VMEM((n_ch, _CH), jnp.int32),
            pltpu.VMEM((rows_w, eb), jnp.float32),
            pltpu.VMEM((rows_w, ec), jnp.float32),
            pltpu.SemaphoreType.DMA,
        ],
        compiler_params=pltpu.CompilerParams(use_tc_tiling_on_sc=False),
    )
    def gather_k(b_hbm, c_hbm, ib_hbm, ic_hbm, bg_hbm, cg_hbm,
                 ib_v, ic_v, rb_v, rc_v, sem):
        wid = lax.axis_index("s") * _NC + lax.axis_index("c")
        base = wid * rows_w
        pltpu.sync_copy(ib_hbm.at[wid], ib_v)
        pltpu.sync_copy(ic_hbm.at[wid], ic_v)
        copies = []
        for j in range(n_ch):
            copies.append(pltpu.async_copy(
                b_hbm.at[ib_v.at[j]], rb_v.at[pl.ds(j * _CH, _CH)], sem))
            copies.append(pltpu.async_copy(
                c_hbm.at[ic_v.at[j]], rc_v.at[pl.ds(j * _CH, _CH)], sem))
        for cp in copies:
            cp.wait()
        pltpu.sync_copy(rb_v, bg_hbm.at[pl.ds(base, rows_w)])
        pltpu.sync_copy(rc_v, cg_hbm.at[pl.ds(base, rows_w)])

    return gather_k(flat_b, flat_c, idx_b, idx_c)


def _tc_body(bg_ref, cg_ref, dt_ref, wb_ref, wc_ref, l1b_ref, a_ref,
             bias_ref, l2_ref, l2b_ref, out_ref):
    bg = bg_ref[...]
    cg = cg_ref[...]
    dt = dt_ref[...]
    rb = jnp.dot(bg, wb_ref[...], preferred_element_type=jnp.float32)
    rc = jnp.dot(cg, wc_ref[...], preferred_element_type=jnp.float32)
    x = jnp.dot(cg, l1b_ref[...], preferred_element_type=jnp.float32)
    x = x + dt * a_ref[...] + bias_ref[...]
    xa = 1.0 / (1.0 + jnp.exp(-x))
    t = jnp.dot(xa, l2_ref[...], preferred_element_type=jnp.float32)
    rate = rb + rc + t + l2b_ref[...]
    out_ref[...] = jnp.maximum(rate, 0.0) + jnp.log1p(jnp.exp(-jnp.abs(rate)))


def kernel(mat_b, mat_c, arr_b_idx, arr_c_idx, arr_delta_t,
           w_b, w_c, lin1a_w, lin1a_b, lin1b_w, lin1b_b, lin2_w, lin2_b):
    B, L1, Eb = mat_b.shape
    _, L2, Ec = mat_c.shape
    L = arr_b_idx.shape[1]
    R = B * L
    rows_w = R // _NW
    n_ch = rows_w // _CH

    ib = arr_b_idx.reshape(-1).astype(jnp.int32)
    ic = arr_c_idx.reshape(-1).astype(jnp.int32)
    boff = jnp.arange(B, dtype=jnp.int32)[:, None]
    gb = (boff * L1 + ib[None, :]).reshape(_NW, n_ch, _CH)
    gc = (boff * L2 + ic[None, :]).reshape(_NW, n_ch, _CH)

    bg, cg = _sc_gather(mat_b.reshape(B * L1, Eb), mat_c.reshape(B * L2, Ec),
                        gb, gc, R, Eb, Ec)

    dt_col = arr_delta_t.astype(jnp.float32).reshape(R, 1)
    wb_col = w_b.reshape(Eb, 1)
    wc_col = w_c.reshape(Ec, 1)
    l1bT = lin1b_w.T
    a_row = lin1a_w.reshape(1, -1)
    bias_row = (lin1a_b + lin1b_b).reshape(1, -1)
    l2_col = lin2_w.reshape(-1, 1)
    l2b = lin2_b.reshape(1, 1)

    RB = 4096
    grid = R // RB
    H = lin1b_w.shape[0]
    out = pl.pallas_call(
        _tc_body,
        grid=(grid,),
        in_specs=[
            pl.BlockSpec((RB, Eb), lambda i: (i, 0)),
            pl.BlockSpec((RB, Ec), lambda i: (i, 0)),
            pl.BlockSpec((RB, 1), lambda i: (i, 0)),
            pl.BlockSpec((Eb, 1), lambda i: (0, 0)),
            pl.BlockSpec((Ec, 1), lambda i: (0, 0)),
            pl.BlockSpec((Ec, H), lambda i: (0, 0)),
            pl.BlockSpec((1, H), lambda i: (0, 0)),
            pl.BlockSpec((1, H), lambda i: (0, 0)),
            pl.BlockSpec((H, 1), lambda i: (0, 0)),
            pl.BlockSpec((1, 1), lambda i: (0, 0)),
        ],
        out_specs=pl.BlockSpec((RB, 1), lambda i: (i, 0)),
        out_shape=jax.ShapeDtypeStruct((R, 1), jnp.float32),
    )(bg, cg, dt_col, wb_col, wc_col, l1bT, a_row, bias_row, l2_col, l2b)

    return out.reshape(B, L)


# speedup vs baseline: 2.6002x; 2.6002x over previous
import functools

import jax
import jax.numpy as jnp
from jax import lax
from jax.experimental import pallas as pl
from jax.experimental.pallas import tpu as pltpu
from jax.experimental.pallas import tpu_sc as plsc

_NC, _NS = 2, 16
_NW = _NC * _NS
_CH = 128


def _sc_gather(tb3, tc3, gq_b, gq_c, sub_b, sub_c, rows, eb, ec):
    rows_w = rows // _NW
    n_ch = rows_w // _CH
    mesh = plsc.VectorSubcoreMesh(core_axis_name="c", subcore_axis_name="s")

    @functools.partial(
        pl.kernel,
        out_type=(jax.ShapeDtypeStruct((rows, eb), jnp.float32),
                  jax.ShapeDtypeStruct((rows, ec), jnp.float32)),
        mesh=mesh,
        scratch_types=[
            pltpu.VMEM((n_ch, _CH), jnp.int32),
            pltpu.VMEM((n_ch, _CH), jnp.int32),
            pltpu.VMEM((rows_w,), jnp.int32),
            pltpu.VMEM((rows_w,), jnp.int32),
            pltpu.VMEM((_CH, 8, eb), jnp.float32),
            pltpu.VMEM((_CH, 8, ec), jnp.float32),
            pltpu.VMEM((_CH, eb), jnp.float32),
            pltpu.VMEM((_CH, ec), jnp.float32),
            pltpu.SemaphoreType.DMA,
        ],
        compiler_params=pltpu.CompilerParams(needs_layout_passes=False),
    )
    def gather_k(tb_h, tc_h, gqb_h, gqc_h, sb_h, sc_h, bg_h, cg_h,
                 gqb_v, gqc_v, sbv, scv, grpb, grpc, stgb, stgc, sem):
        wid = lax.axis_index("s") * _NC + lax.axis_index("c")
        base = wid * rows_w
        pltpu.sync_copy(gqb_h.at[wid], gqb_v)
        pltpu.sync_copy(gqc_h.at[wid], gqc_v)
        pltpu.sync_copy(sb_h.at[wid], sbv)
        pltpu.sync_copy(sc_h.at[wid], scv)

        @pl.loop(0, n_ch)
        def chunk(ch):
            cps = []
            for g in range(_CH // 16):
                qbv = gqb_v[ch, pl.ds(g * 16, 16)]
                qcv = gqc_v[ch, pl.ds(g * 16, 16)]
                for j in range(16):
                    k = g * 16 + j
                    cps.append(pltpu.async_copy(
                        tb_h.at[qbv[j]], stgb.at[k], sem))
                    cps.append(pltpu.async_copy(
                        tc_h.at[qcv[j]], stgc.at[k], sem))
            for cp in cps:
                cp.wait()
            pltpu.sync_copy(stgb, bg_h.at[pl.ds(base + ch * _CH, _CH)])
            pltpu.sync_copy(stgc, cg_h.at[pl.ds(base + ch * _CH, _CH)])

    return gather_k(tb3, tc3, gq_b, gq_c, sub_b, sub_c)


def _tc_body(bg_ref, cg_ref, dt_ref, wb_ref, wc_ref, l1b_ref, a_ref,
             bias_ref, l2_ref, l2b_ref, out_ref):
    bg = bg_ref[...]
    cg = cg_ref[...]
    dt = dt_ref[...]
    rb = jnp.dot(bg, wb_ref[...], preferred_element_type=jnp.float32)
    rc = jnp.dot(cg, wc_ref[...], preferred_element_type=jnp.float32)
    x = jnp.dot(cg, l1b_ref[...], preferred_element_type=jnp.float32)
    x = x + dt * a_ref[...] + bias_ref[...]
    xa = 1.0 / (1.0 + jnp.exp(-x))
    t = jnp.dot(xa, l2_ref[...], preferred_element_type=jnp.float32)
    rate = rb + rc + t + l2b_ref[...]
    out_ref[...] = jnp.maximum(rate, 0.0) + jnp.log1p(jnp.exp(-jnp.abs(rate)))


def kernel(mat_b, mat_c, arr_b_idx, arr_c_idx, arr_delta_t,
           w_b, w_c, lin1a_w, lin1a_b, lin1b_w, lin1b_b, lin2_w, lin2_b):
    B, L1, Eb = mat_b.shape
    _, L2, Ec = mat_c.shape
    L = arr_b_idx.shape[1]
    R = B * L
    rows_w = R // _NW
    n_ch = rows_w // _CH

    ib = arr_b_idx.reshape(-1).astype(jnp.int32)
    ic = arr_c_idx.reshape(-1).astype(jnp.int32)
    boff = jnp.arange(B, dtype=jnp.int32)[:, None]
    gb = boff * L1 + ib[None, :]
    gc = boff * L2 + ic[None, :]
    gq_b = gb.reshape(_NW, n_ch, _CH)
    gq_c = gc.reshape(_NW, n_ch, _CH)
    sub_b = (gb & 7).reshape(_NW, rows_w)
    sub_c = (gc & 7).reshape(_NW, rows_w)

    bg, cg = _sc_gather(mat_b.reshape(B * L1, Eb),
                        mat_c.reshape(B * L2, Ec),
                        gq_b, gq_c, sub_b, sub_c, R, Eb, Ec)

    dt_col = arr_delta_t.astype(jnp.float32).reshape(R, 1)
    wb_col = w_b.reshape(Eb, 1)
    wc_col = w_c.reshape(Ec, 1)
    l1bT = lin1b_w.T
    a_row = lin1a_w.reshape(1, -1)
    bias_row = (lin1a_b + lin1b_b).reshape(1, -1)
    l2_col = lin2_w.reshape(-1, 1)
    l2b = lin2_b.reshape(1, 1)

    RB = 4096
    grid = R // RB
    H = lin1b_w.shape[0]
    out = pl.pallas_call(
        _tc_body,
        grid=(grid,),
        in_specs=[
            pl.BlockSpec((RB, Eb), lambda i: (i, 0)),
            pl.BlockSpec((RB, Ec), lambda i: (i, 0)),
            pl.BlockSpec((RB, 1), lambda i: (i, 0)),
            pl.BlockSpec((Eb, 1), lambda i: (0, 0)),
            pl.BlockSpec((Ec, 1), lambda i: (0, 0)),
            pl.BlockSpec((Ec, H), lambda i: (0, 0)),
            pl.BlockSpec((1, H), lambda i: (0, 0)),
            pl.BlockSpec((1, H), lambda i: (0, 0)),
            pl.BlockSpec((H, 1), lambda i: (0, 0)),
            pl.BlockSpec((1, 1), lambda i: (0, 0)),
        ],
        out_specs=pl.BlockSpec((RB, 1), lambda i: (i, 0)),
        out_shape=jax.ShapeDtypeStruct((R, 1), jnp.float32),
    )(bg, cg, dt_col, wb_col, wc_col, l1bT, a_row, bias_row, l2_col, l2b)

    return out.reshape(B, L)
